# baseline (speedup 1.0000x reference)
import jax
import jax.numpy as jnp
from jax import lax
from jax.experimental import pallas as pl
from jax.experimental.pallas import tpu as pltpu

N_DOMAIN = 8
EPS = 1e-05
ROWS = 16384
COLS = 1024
BR = 2048
NB = ROWS // BR
COLH = 512
NH = COLS // COLH


def _onehot_t(y_ref, i):
    yv = y_ref[i]
    ids = lax.broadcasted_iota(jnp.int32, (N_DOMAIN, BR), 0)
    return (ids == yv).astype(jnp.float32)


def _pipe_kernel(y_ref, g_ref, b_ref, x_any, out_ref,
                 stg0, stg1, xb0, xb1, st0, st1, cnt, sems):
    s = pl.program_id(0)
    i = pl.program_id(1)

    def read(blk, h, stg_ref, slot):
        return pltpu.make_async_copy(
            x_any.at[pl.ds(blk * BR, BR), pl.ds(h * COLH, COLH)],
            stg_ref, sems.at[slot])

    def issue(blk, h):
        @pl.when(blk % 2 == 0)
        def _e():
            read(blk, h, stg0, 0).start()

        @pl.when(blk % 2 == 1)
        def _o():
            read(blk, h, stg1, 1).start()

    oh = _onehot_t(y_ref, i)

    @pl.when(s < NH)
    def _stats():
        @pl.when(jnp.logical_and(s == 0, i == 0))
        def _first():
            issue(0, 0)

        @pl.when(i + 1 < NB)
        def _next():
            issue(i + 1, s)

        @pl.when(jnp.logical_and(i == NB - 1, s + 1 < NH))
        def _nextstage():
            issue(0, s + 1)

        def accum(stg_ref, slot, sums_ref, sumsq_ref, xbuf):
            read(i, s, stg_ref, slot).wait()

            @pl.when(i == 0)
            def _zero():
                sums_ref[...] = jnp.zeros_like(sums_ref)
                sumsq_ref[...] = jnp.zeros_like(sumsq_ref)

            xb = stg_ref[...]
            sums_ref[...] += lax.dot_general(
                oh, xb, (((1,), (0,)), ((), ())),
                preferred_element_type=jnp.float32)
            sumsq_ref[...] += lax.dot_general(
                oh, xb * xb, (((1,), (0,)), ((), ())),
                preferred_element_type=jnp.float32)
            xbuf[pl.ds(i * BR, BR), :] = xb.astype(jnp.bfloat16)

        def accum_buf(sums_ref, sumsq_ref, xbuf):
            @pl.when(i % 2 == 0)
            def _e():
                accum(stg0, 0, sums_ref, sumsq_ref, xbuf)

            @pl.when(i % 2 == 1)
            def _o():
                accum(stg1, 1, sums_ref, sumsq_ref, xbuf)

        @pl.when(s == 0)
        def _h0():
            accum_buf(st0[0], st0[1], xb0)

            @pl.when(i == 0)
            def _zc():
                cnt[...] = jnp.zeros_like(cnt)

            cnt[...] += jnp.broadcast_to(
                jnp.sum(oh, axis=1, keepdims=True), cnt.shape)

        @pl.when(s == 1)
        def _h1():
            accum_buf(st1[0], st1[1], xb1)

    @pl.when(s >= 1)
    def _apply():
        h = s - 1

        def apply_from(st, xbuf):
            @pl.when(i == 0)
            def _tables():
                c = cnt[:, :1]
                denom = jnp.maximum(c, 1.0)
                mean = st[0][...] / denom
                var = jnp.maximum(st[1][...] / denom - mean * mean, 0.0)
                gh = g_ref[:, pl.ds(h * COLH, COLH)]
                bh = b_ref[:, pl.ds(h * COLH, COLH)]
                scale = gh * lax.rsqrt(var + EPS)
                multi = c > 1.0
                st[2][...] = jnp.where(multi, scale, 1.0)
                st[3][...] = jnp.where(multi, bh - mean * scale, 0.0)

            row_a = lax.dot_general(oh, st[2][...], (((0,), (0,)), ((), ())),
                                    preferred_element_type=jnp.float32)
            row_b = lax.dot_general(oh, st[3][...], (((0,), (0,)), ((), ())),
                                    preferred_element_type=jnp.float32)
            xb = xbuf[pl.ds(i * BR, BR), :].astype(jnp.float32)
            out_ref[...] = xb * row_a + row_b

        @pl.when(s == 1)
        def _a0():
            apply_from(st0, xb0)

        @pl.when(s == 2)
        def _a1():
            apply_from(st1, xb1)


@jax.jit
def kernel(x, y, gamma, beta):
    y3 = y.astype(jnp.int32).reshape(NB, 1, BR)
    stat = lambda: pltpu.VMEM((N_DOMAIN, COLH), jnp.float32)
    out = pl.pallas_call(
        _pipe_kernel,
        grid=(NH + 1, NB),
        in_specs=[
            pl.BlockSpec((NB, 1, BR), lambda s, i: (0, 0, 0)),
            pl.BlockSpec((1, COLS), lambda s, i: (0, 0)),
            pl.BlockSpec((1, COLS), lambda s, i: (0, 0)),
            pl.BlockSpec(memory_space=pl.ANY),
        ],
        out_specs=pl.BlockSpec(
            (BR, COLH),
            lambda s, i: (jnp.where(s > 0, i, 0), jnp.maximum(s - 1, 0))),
        out_shape=jax.ShapeDtypeStruct((ROWS, COLS), jnp.float32),
        scratch_shapes=[
            pltpu.VMEM((BR, COLH), jnp.float32),
            pltpu.VMEM((BR, COLH), jnp.float32),
            pltpu.VMEM((ROWS, COLH), jnp.bfloat16),
            pltpu.VMEM((ROWS, COLH), jnp.bfloat16),
            [stat(), stat(), stat(), stat()],
            [stat(), stat(), stat(), stat()],
            pltpu.VMEM((N_DOMAIN, 128), jnp.float32),
            pltpu.SemaphoreType.DMA((2,)),
        ],
    )(y3, gamma, beta, x)
    return out


# speedup vs baseline: 1.0367x; 1.0367x over previous
import jax
import jax.numpy as jnp
from jax import lax
from jax.experimental import pallas as pl
from jax.experimental.pallas import tpu as pltpu

N_DOMAIN = 8
EPS = 1e-05
ROWS = 16384
COLS = 1024
BR = 2048
NB = ROWS // BR
COLH = 512
NH = COLS // COLH


def _onehot_t(y_ref, i):
    yv = y_ref[i]
    ids = lax.broadcasted_iota(jnp.int32, (N_DOMAIN, BR), 0)
    return (ids == yv).astype(jnp.float32)


def _kernel(y_ref, g_ref, b_ref, x_any, out_any,
            xbuf, ob0, ob1, sums, sumsq, cnt, atab, btab, rs, ws):
    h = pl.program_id(0)
    p = pl.program_id(1)

    def rd(blk, hh):
        return pltpu.make_async_copy(
            x_any.at[pl.ds(blk * BR, BR), pl.ds(hh * COLH, COLH)],
            xbuf.at[pl.ds(blk * BR, BR), :], rs.at[blk % 2])

    def wr(blk, obuf):
        return pltpu.make_async_copy(
            obuf, out_any.at[pl.ds(blk * BR, BR), pl.ds(h * COLH, COLH)],
            ws.at[blk % 2])

    @pl.when(p == 0)
    def _phase0():
        @pl.when(h == 0)
        def _prime():
            rd(0, h).start()
            rd(1, h).start()

        sums[...] = jnp.zeros_like(sums)
        sumsq[...] = jnp.zeros_like(sumsq)

        @pl.when(h == 0)
        def _zc():
            cnt[...] = jnp.zeros_like(cnt)

        for i in range(NB):
            rd(i, h).wait()
            if i + 2 < NB:
                rd(i + 2, h).start()
            xb = xbuf[pl.ds(i * BR, BR), :]
            oh = _onehot_t(y_ref, i)
            sums[...] += lax.dot_general(
                oh, xb, (((1,), (0,)), ((), ())),
                preferred_element_type=jnp.float32)
            sumsq[...] += lax.dot_general(
                oh, xb * xb, (((1,), (0,)), ((), ())),
                preferred_element_type=jnp.float32)

            @pl.when(h == 0)
            def _count():
                cnt[...] += jnp.broadcast_to(
                    jnp.sum(oh, axis=1, keepdims=True), cnt.shape)

    @pl.when(p == 1)
    def _phase1():
        c = cnt[:, :1]
        denom = jnp.maximum(c, 1.0)
        mean = sums[...] / denom
        var = jnp.maximum(sumsq[...] / denom - mean * mean, 0.0)
        gh = g_ref[:, pl.ds(h * COLH, COLH)]
        bh = b_ref[:, pl.ds(h * COLH, COLH)]
        scale = gh * lax.rsqrt(var + EPS)
        multi = c > 1.0
        atab[...] = jnp.where(multi, scale, 1.0)
        btab[...] = jnp.where(multi, bh - mean * scale, 0.0)

        for i in range(NB):
            obuf = ob0 if i % 2 == 0 else ob1
            if i >= 2:
                wr(i - 2, obuf).wait()
            oh = _onehot_t(y_ref, i)
            row_a = lax.dot_general(
                oh, atab[...], (((0,), (0,)), ((), ())),
                preferred_element_type=jnp.float32)
            row_b = lax.dot_general(
                oh, btab[...], (((0,), (0,)), ((), ())),
                preferred_element_type=jnp.float32)
            obuf[...] = xbuf[pl.ds(i * BR, BR), :] * row_a + row_b
            wr(i, obuf).start()

        @pl.when(h + 1 < NH)
        def _prefetch():
            rd(0, h + 1).start()
            rd(1, h + 1).start()

        wr(NB - 2, ob0 if (NB - 2) % 2 == 0 else ob1).wait()
        wr(NB - 1, ob0 if (NB - 1) % 2 == 0 else ob1).wait()


@jax.jit
def kernel(x, y, gamma, beta):
    y3 = y.astype(jnp.int32).reshape(NB, 1, BR)
    out = pl.pallas_call(
        _kernel,
        grid=(NH, 2),
        in_specs=[
            pl.BlockSpec((NB, 1, BR), lambda h, p: (0, 0, 0)),
            pl.BlockSpec((1, COLS), lambda h, p: (0, 0)),
            pl.BlockSpec((1, COLS), lambda h, p: (0, 0)),
            pl.BlockSpec(memory_space=pl.ANY),
        ],
        out_specs=pl.BlockSpec(memory_space=pl.ANY),
        out_shape=jax.ShapeDtypeStruct((ROWS, COLS), jnp.float32),
        scratch_shapes=[
            pltpu.VMEM((ROWS, COLH), jnp.float32),
            pltpu.VMEM((BR, COLH), jnp.float32),
            pltpu.VMEM((BR, COLH), jnp.float32),
            pltpu.VMEM((N_DOMAIN, COLH), jnp.float32),
            pltpu.VMEM((N_DOMAIN, COLH), jnp.float32),
            pltpu.VMEM((N_DOMAIN, 128), jnp.float32),
            pltpu.VMEM((N_DOMAIN, COLH), jnp.float32),
            pltpu.VMEM((N_DOMAIN, COLH), jnp.float32),
            pltpu.SemaphoreType.DMA((2,)),
            pltpu.SemaphoreType.DMA((2,)),
        ],
    )(y3, gamma, beta, x)
    return out


# speedup vs baseline: 1.1266x; 1.0867x over previous
import jax
import jax.numpy as jnp
from jax import lax
from jax.experimental import pallas as pl
from jax.experimental.pallas import tpu as pltpu

N_DOMAIN = 8
EPS = 1e-05
ROWS = 16384
COLS = 1024
BR = 2048
NB = ROWS // BR
COLH = 512
NH = COLS // COLH


def _onehot_t(y_ref, i):
    yv = y_ref[i]
    ids = lax.broadcasted_iota(jnp.int32, (N_DOMAIN, BR), 0)
    return (ids == yv).astype(jnp.float32)


def _kernel(y_ref, g_ref, b_ref, x_any, out_any,
            xbuf, ob0, ob1, ob2, ob3, sums, sumsq, cnt, atab, btab, rs, ws):
    obs = [ob0, ob1, ob2, ob3]
    h = pl.program_id(0)
    p = pl.program_id(1)

    def rd(blk, hh):
        return pltpu.make_async_copy(
            x_any.at[pl.ds(blk * BR, BR), pl.ds(hh * COLH, COLH)],
            xbuf.at[pl.ds(blk * BR, BR), :], rs.at[blk % 4])

    def wr(blk, obuf):
        return pltpu.make_async_copy(
            obuf, out_any.at[pl.ds(blk * BR, BR), pl.ds(h * COLH, COLH)],
            ws.at[blk % 4])

    @pl.when(p == 0)
    def _phase0():
        @pl.when(h == 0)
        def _prime():
            rd(0, h).start()
            rd(1, h).start()
            rd(2, h).start()
            rd(3, h).start()

        sums[...] = jnp.zeros_like(sums)
        sumsq[...] = jnp.zeros_like(sumsq)

        @pl.when(h == 0)
        def _zc():
            cnt[...] = jnp.zeros_like(cnt)

        for i in range(NB):
            rd(i, h).wait()
            if i + 4 < NB:
                rd(i + 4, h).start()
            xb = xbuf[pl.ds(i * BR, BR), :]
            oh = _onehot_t(y_ref, i)
            sums[...] += lax.dot_general(
                oh, xb, (((1,), (0,)), ((), ())),
                preferred_element_type=jnp.float32)
            sumsq[...] += lax.dot_general(
                oh, xb * xb, (((1,), (0,)), ((), ())),
                preferred_element_type=jnp.float32)

            @pl.when(h == 0)
            def _count():
                cnt[...] += jnp.broadcast_to(
                    jnp.sum(oh, axis=1, keepdims=True), cnt.shape)

    @pl.when(p == 1)
    def _phase1():
        c = cnt[:, :1]
        denom = jnp.maximum(c, 1.0)
        mean = sums[...] / denom
        var = jnp.maximum(sumsq[...] / denom - mean * mean, 0.0)
        gh = g_ref[:, pl.ds(h * COLH, COLH)]
        bh = b_ref[:, pl.ds(h * COLH, COLH)]
        scale = gh * lax.rsqrt(var + EPS)
        multi = c > 1.0
        atab[...] = jnp.where(multi, scale, 1.0)
        btab[...] = jnp.where(multi, bh - mean * scale, 0.0)

        for i in range(NB):
            obuf = obs[i % 4]
            if i >= 4:
                wr(i - 4, obuf).wait()
            oh = _onehot_t(y_ref, i)
            row_a = lax.dot_general(
                oh, atab[...], (((0,), (0,)), ((), ())),
                preferred_element_type=jnp.float32)
            row_b = lax.dot_general(
                oh, btab[...], (((0,), (0,)), ((), ())),
                preferred_element_type=jnp.float32)
            obuf[...] = xbuf[pl.ds(i * BR, BR), :] * row_a + row_b
            wr(i, obuf).start()

        @pl.when(h + 1 < NH)
        def _prefetch():
            rd(0, h + 1).start()
            rd(1, h + 1).start()
            rd(2, h + 1).start()
            rd(3, h + 1).start()

        for j in range(NB - 4, NB):
            wr(j, obs[j % 4]).wait()


@jax.jit
def kernel(x, y, gamma, beta):
    y3 = y.astype(jnp.int32).reshape(NB, 1, BR)
    out = pl.pallas_call(
        _kernel,
        grid=(NH, 2),
        in_specs=[
            pl.BlockSpec((NB, 1, BR), lambda h, p: (0, 0, 0)),
            pl.BlockSpec((1, COLS), lambda h, p: (0, 0)),
            pl.BlockSpec((1, COLS), lambda h, p: (0, 0)),
            pl.BlockSpec(memory_space=pl.ANY),
        ],
        out_specs=pl.BlockSpec(memory_space=pl.ANY),
        out_shape=jax.ShapeDtypeStruct((ROWS, COLS), jnp.float32),
        scratch_shapes=[
            pltpu.VMEM((ROWS, COLH), jnp.float32),
            pltpu.VMEM((BR, COLH), jnp.float32),
            pltpu.VMEM((BR, COLH), jnp.float32),
            pltpu.VMEM((BR, COLH), jnp.float32),
            pltpu.VMEM((BR, COLH), jnp.float32),
            pltpu.VMEM((N_DOMAIN, COLH), jnp.float32),
            pltpu.VMEM((N_DOMAIN, COLH), jnp.float32),
            pltpu.VMEM((N_DOMAIN, 128), jnp.float32),
            pltpu.VMEM((N_DOMAIN, COLH), jnp.float32),
            pltpu.VMEM((N_DOMAIN, COLH), jnp.float32),
            pltpu.SemaphoreType.DMA((4,)),
            pltpu.SemaphoreType.DMA((4,)),
        ],
    )(y3, gamma, beta, x)
    return out


# speedup vs baseline: 1.1516x; 1.0221x over previous
import jax
import jax.numpy as jnp
from jax import lax
from jax.experimental import pallas as pl
from jax.experimental.pallas import tpu as pltpu

N_DOMAIN = 8
EPS = 1e-05
ROWS = 16384
COLS = 1024
BR = 1024
NB = ROWS // BR
COLH = 512
NH = COLS // COLH
NWB = 8


def _onehot_t(y_ref, i):
    yv = y_ref[i]
    ids = lax.broadcasted_iota(jnp.int32, (N_DOMAIN, BR), 0)
    return (ids == yv).astype(jnp.float32)


def _kernel(y_ref, g_ref, b_ref, x_any, out_any,
            xbuf, ob0, ob1, ob2, ob3, ob4, ob5, ob6, ob7,
            sums, sumsq, cnt, atab, btab, rs, ws):
    h = pl.program_id(0)
    p = pl.program_id(1)
    obs = [ob0, ob1, ob2, ob3, ob4, ob5, ob6, ob7]

    def rd(blk, hh):
        return pltpu.make_async_copy(
            x_any.at[pl.ds(blk * BR, BR), pl.ds(hh * COLH, COLH)],
            xbuf.at[pl.ds(blk * BR, BR), :], rs.at[blk])

    def wr(blk, obuf):
        return pltpu.make_async_copy(
            obuf, out_any.at[pl.ds(blk * BR, BR), pl.ds(h * COLH, COLH)],
            ws.at[blk % NWB])

    @pl.when(p == 0)
    def _phase0():
        @pl.when(h == 0)
        def _prime():
            for j in range(NB):
                rd(j, h).start()

        sums[...] = jnp.zeros_like(sums)
        sumsq[...] = jnp.zeros_like(sumsq)

        @pl.when(h == 0)
        def _zc():
            cnt[...] = jnp.zeros_like(cnt)

        for i in range(NB):
            rd(i, h).wait()
            xb = xbuf[pl.ds(i * BR, BR), :]
            oh = _onehot_t(y_ref, i)
            sums[...] += lax.dot_general(
                oh, xb, (((1,), (0,)), ((), ())),
                preferred_element_type=jnp.float32)
            sumsq[...] += lax.dot_general(
                oh, xb * xb, (((1,), (0,)), ((), ())),
                preferred_element_type=jnp.float32)

            @pl.when(h == 0)
            def _count():
                cnt[...] += jnp.broadcast_to(
                    jnp.sum(oh, axis=1, keepdims=True), cnt.shape)

    @pl.when(p == 1)
    def _phase1():
        c = cnt[:, :1]
        denom = jnp.maximum(c, 1.0)
        mean = sums[...] / denom
        var = jnp.maximum(sumsq[...] / denom - mean * mean, 0.0)
        gh = g_ref[:, pl.ds(h * COLH, COLH)]
        bh = b_ref[:, pl.ds(h * COLH, COLH)]
        scale = gh * lax.rsqrt(var + EPS)
        multi = c > 1.0
        atab[...] = jnp.where(multi, scale, 1.0)
        btab[...] = jnp.where(multi, bh - mean * scale, 0.0)

        for i in range(NB):
            obuf = obs[i % NWB]
            if i >= NWB:
                wr(i - NWB, obuf).wait()
            oh = _onehot_t(y_ref, i)
            row_a = lax.dot_general(
                oh, atab[...], (((0,), (0,)), ((), ())),
                preferred_element_type=jnp.float32)
            row_b = lax.dot_general(
                oh, btab[...], (((0,), (0,)), ((), ())),
                preferred_element_type=jnp.float32)
            obuf[...] = xbuf[pl.ds(i * BR, BR), :] * row_a + row_b
            wr(i, obuf).start()

        @pl.when(h + 1 < NH)
        def _prefetch():
            for j in range(NB):
                rd(j, h + 1).start()

        for j in range(NB - NWB, NB):
            wr(j, obs[j % NWB]).wait()


@jax.jit
def kernel(x, y, gamma, beta):
    y3 = y.astype(jnp.int32).reshape(NB, 1, BR)
    out = pl.pallas_call(
        _kernel,
        grid=(NH, 2),
        in_specs=[
            pl.BlockSpec((NB, 1, BR), lambda h, p: (0, 0, 0)),
            pl.BlockSpec((1, COLS), lambda h, p: (0, 0)),
            pl.BlockSpec((1, COLS), lambda h, p: (0, 0)),
            pl.BlockSpec(memory_space=pl.ANY),
        ],
        out_specs=pl.BlockSpec(memory_space=pl.ANY),
        out_shape=jax.ShapeDtypeStruct((ROWS, COLS), jnp.float32),
        scratch_shapes=[
            pltpu.VMEM((ROWS, COLH), jnp.float32),
        ] + [pltpu.VMEM((BR, COLH), jnp.float32) for _ in range(8)] + [
            pltpu.VMEM((N_DOMAIN, COLH), jnp.float32),
            pltpu.VMEM((N_DOMAIN, COLH), jnp.float32),
            pltpu.VMEM((N_DOMAIN, 128), jnp.float32),
            pltpu.VMEM((N_DOMAIN, COLH), jnp.float32),
            pltpu.VMEM((N_DOMAIN, COLH), jnp.float32),
            pltpu.SemaphoreType.DMA((NB,)),
            pltpu.SemaphoreType.DMA((NWB,)),
        ],
    )(y3, gamma, beta, x)
    return out
